# baseline (speedup 1.0000x reference)
import functools

import jax
import jax.numpy as jnp
from jax import lax
from jax.experimental import pallas as pl
from jax.experimental.pallas import tpu as pltpu
from jax.experimental.pallas import tpu_sc as plsc

_NW = 32
_SUB = 128
_LANES = 16
_BLC = 256


def _sc_gather_call(fdi, fti, fd_tab, ft_tab):
    n = fdi.shape[0]
    nch = n // _SUB
    rows_w = n // _NW
    ch_w = rows_w // _SUB
    mesh = plsc.VectorSubcoreMesh(core_axis_name="c", subcore_axis_name="s")

    @functools.partial(
        pl.kernel,
        out_type=jax.ShapeDtypeStruct((nch, 48, _SUB), jnp.float32),
        mesh=mesh,
        compiler_params=pltpu.CompilerParams(use_tc_tiling_on_sc=False,
                                             needs_layout_passes=False),
        scratch_types=[
            pltpu.VMEM_SHARED((2936, 16), jnp.float32),
            pltpu.VMEM_SHARED((21615, 32), jnp.float32),
            pltpu.VMEM((rows_w,), jnp.int32),
            pltpu.VMEM((rows_w,), jnp.int32),
            pltpu.VMEM((2, _SUB, 16), jnp.float32),
            pltpu.VMEM((2, _SUB, 32), jnp.float32),
            pltpu.VMEM((2, 48, _SUB), jnp.float32),
            pltpu.SemaphoreType.DMA,
            pltpu.SemaphoreType.DMA,
            pltpu.SemaphoreType.DMA,
            pltpu.SemaphoreType.DMA,
            pltpu.SemaphoreType.DMA,
            pltpu.SemaphoreType.DMA,
            pltpu.SemaphoreType.DMA,
        ],
    )
    def sc_kernel(fdi_h, fti_h, fd_h, ft_h, x_h,
                  fd_s, ft_s, fdi_v, fti_v, bfd, bft, tb,
                  semi, semgf0, semgf1, semgt0, semgt1, semw0, semw1):
        iota = lax.iota(jnp.int32, _LANES)
        rows_l = [iota + (g * _LANES) for g in range(_SUB // _LANES)]
        wid = lax.axis_index("s") * 2 + lax.axis_index("c")
        base_c = wid * ch_w
        semgf = (semgf0, semgf1)
        semgt = (semgt0, semgt1)
        semw = (semw0, semw1)

        @pl.when(lax.axis_index("s") == 0)
        def _():
            pltpu.sync_copy(fd_h, fd_s)
            pltpu.sync_copy(ft_h, ft_s)

        pltpu.async_copy(fdi_h.at[pl.ds(wid * rows_w, rows_w)], fdi_v, semi).wait()
        pltpu.async_copy(fti_h.at[pl.ds(wid * rows_w, rows_w)], fti_v, semi).wait()
        plsc.subcore_barrier()

        def issue(ci, b):
            s = pl.ds(ci * _SUB, _SUB)
            pltpu.async_copy(fd_s.at[fdi_v.at[s]], bfd.at[b], semgf[b])
            pltpu.async_copy(ft_s.at[fti_v.at[s]], bft.at[b], semgt[b])

        def wait_gather(ci, b):
            s = pl.ds(ci * _SUB, _SUB)
            pltpu.make_async_copy(fd_s.at[fdi_v.at[s]], bfd.at[b], semgf[b]).wait()
            pltpu.make_async_copy(ft_s.at[fti_v.at[s]], bft.at[b], semgt[b]).wait()

        def wait_wb(ci, b):
            pltpu.make_async_copy(tb.at[b], x_h.at[base_c + ci], semw[b]).wait()

        zero = jnp.zeros((_LANES,), jnp.float32)
        for b in range(2):
            for f in list(range(13, 16)) + list(range(41, 48)):
                for g in range(_SUB // _LANES):
                    tb[b, f, pl.ds(g * _LANES, _LANES)] = zero

        issue(0, 0)
        issue(1, 1)

        def pair(pi, carry):
            for b in range(2):
                ci = pi * 2 + b
                wait_gather(ci, b)

                @pl.when(ci >= 2)
                def _():
                    wait_wb(ci - 2, b)

                for f in range(13):
                    cols = jnp.full((_LANES,), f, jnp.int32)
                    vs = [plsc.load_gather(bfd.at[b], [rows_l[g], cols])
                          for g in range(_SUB // _LANES)]
                    for g in range(_SUB // _LANES):
                        tb[b, f, pl.ds(g * _LANES, _LANES)] = vs[g]
                for f in range(25):
                    cols = jnp.full((_LANES,), f, jnp.int32)
                    vs = [plsc.load_gather(bft.at[b], [rows_l[g], cols])
                          for g in range(_SUB // _LANES)]
                    for g in range(_SUB // _LANES):
                        tb[b, 16 + f, pl.ds(g * _LANES, _LANES)] = vs[g]

                pltpu.async_copy(tb.at[b], x_h.at[base_c + ci], semw[b])

                @pl.when(ci + 2 < ch_w)
                def _():
                    issue(ci + 2, b)
            return carry

        lax.fori_loop(0, ch_w // 2, pair, 0)
        wait_wb(ch_w - 2, 0)
        wait_wb(ch_w - 1, 1)

    return sc_kernel(fdi, fti, fd_tab, ft_tab)


def _tc_body(x_ref, f1_ref, f2_ref, a1_ref, a2_ref, w2_ref,
             b1_ref, b2_ref, feat_ref, wk_ref, dr_ref, tr_ref):
    r = _BLC * _SUB
    x = x_ref[...].transpose(1, 0, 2).reshape(48, r)
    x1 = x[0:16]
    x2 = x[16:48]
    h1 = (jnp.dot(a1_ref[...], x1, preferred_element_type=jnp.float32)
          + jnp.dot(a2_ref[...], x2, preferred_element_type=jnp.float32)
          + b1_ref[...])
    a = jnp.maximum(h1, 0.01 * h1)
    h2 = jnp.dot(w2_ref[...], a, preferred_element_type=jnp.float32) + b2_ref[...]
    nl = _BLC // 32
    feat_ref[:, 0:1, :] = f1_ref[...].reshape(1, nl, 4096).transpose(1, 0, 2)
    feat_ref[:, 1:2, :] = f2_ref[...].reshape(1, nl, 4096).transpose(1, 0, 2)
    feat_ref[:, 2:7, :] = x2[20:25].reshape(5, nl, 4096).transpose(1, 0, 2)
    feat_ref[:, 7:20, :] = (h2[0:13] + x1[0:13]).reshape(13, nl, 4096).transpose(1, 0, 2)
    feat_ref[:, 20:40, :] = (h2[13:33] + x2[0:20]).reshape(20, nl, 4096).transpose(1, 0, 2)
    wk_ref[...] = x1[0:3].reshape(3, nl, 4096)
    dr_ref[...] = x1[3:13].reshape(10, nl, 4096)
    tr_ref[...] = x2[0:20].reshape(20, nl, 4096)


def _tc_call(x, f1t, f2t, a1, a2, w2, b1c, b2c):
    nch = x.shape[0]
    grid = (nch // _BLC,)
    nl = _BLC // 32
    full = lambda a: pl.BlockSpec(a.shape, lambda i: (0,) * a.ndim)
    out3 = lambda d: pl.BlockSpec((d, nl, 4096), lambda i: (0, i, 0))
    feat_spec = pl.BlockSpec((nl, 40, 4096), lambda i: (i, 0, 0))
    return pl.pallas_call(
        _tc_body,
        grid=grid,
        in_specs=[
            pl.BlockSpec((_BLC, 48, _SUB), lambda i: (i, 0, 0)),
            pl.BlockSpec((nl, 4096), lambda i: (i, 0)),
            pl.BlockSpec((nl, 4096), lambda i: (i, 0)),
            full(a1), full(a2), full(w2), full(b1c), full(b2c),
        ],
        out_specs=[feat_spec, out3(3), out3(10), out3(20)],
        out_shape=[
            jax.ShapeDtypeStruct((200, 40, 4096), jnp.float32),
            jax.ShapeDtypeStruct((3, 200, 4096), jnp.float32),
            jax.ShapeDtypeStruct((10, 200, 4096), jnp.float32),
            jax.ShapeDtypeStruct((20, 200, 4096), jnp.float32),
        ],
    )(x, f1t, f2t, a1, a2, w2, b1c, b2c)


def kernel(links, hw_tab, wk_tab, dt_tab, tm_tab, W1, b1, W2, b2, args):
    bb, ll, _ = links.shape
    n = bb * ll

    i0 = links[:, :, 0].astype(jnp.int32).T.reshape(n)
    i3 = links[:, :, 3].astype(jnp.int32).T.reshape(n)
    i4 = links[:, :, 4].astype(jnp.int32).T.reshape(n)
    i5 = links[:, :, 5].astype(jnp.int32).T.reshape(n)
    f1t = links[:, :, 1].T
    f2t = links[:, :, 2].T
    fdi = i3 * 367 + i4
    fti = i0 * 1441 + i5

    hw_z = hw_tab.at[0].set(0.0)
    nw, nd = wk_tab.shape[0], dt_tab.shape[0]
    nh, nt = hw_tab.shape[0], tm_tab.shape[0]
    fd_tab = jnp.concatenate([
        jnp.broadcast_to(wk_tab[:, None, :], (nw, nd, 3)),
        jnp.broadcast_to(dt_tab[None, :, :], (nw, nd, 10)),
        jnp.zeros((nw, nd, 3), jnp.float32),
    ], axis=-1).reshape(nw * nd, 16)
    ft_tab = jnp.concatenate([
        jnp.broadcast_to(tm_tab[None, :, :], (nh, nt, 20)),
        jnp.broadcast_to(hw_z[:, None, :], (nh, nt, 5)),
        jnp.zeros((nh, nt, 7), jnp.float32),
    ], axis=-1).reshape(nh * nt, 32)

    x = _sc_gather_call(fdi, fti, fd_tab, ft_tab)

    a1 = jnp.concatenate([W1[:, 0:13], jnp.zeros((33, 3), jnp.float32)], 1)
    a2 = jnp.concatenate([W1[:, 13:33], jnp.zeros((33, 12), jnp.float32)], 1)
    featt, wkt, drt, trt = _tc_call(x, f1t, f2t, a1, a2, W2,
                                    b1.reshape(33, 1), b2.reshape(33, 1))
    return (featt.transpose(2, 0, 1),
            (wkt.transpose(2, 1, 0), drt.transpose(2, 1, 0),
             trt.transpose(2, 1, 0)))


# speedup vs baseline: 1.0071x; 1.0071x over previous
import functools

import jax
import jax.numpy as jnp
from jax import lax
from jax.experimental import pallas as pl
from jax.experimental.pallas import tpu as pltpu
from jax.experimental.pallas import tpu_sc as plsc

_NW = 32
_SUB = 128
_LANES = 16
_BLC = 256


def _sc_gather_call(fdi, fti, fd_tab, ft_tab):
    n = fdi.shape[0]
    nch = n // _SUB
    rows_w = n // _NW
    ch_w = rows_w // _SUB
    mesh = plsc.VectorSubcoreMesh(core_axis_name="c", subcore_axis_name="s")

    @functools.partial(
        pl.kernel,
        out_type=jax.ShapeDtypeStruct((nch, 48, _SUB), jnp.float32),
        mesh=mesh,
        compiler_params=pltpu.CompilerParams(use_tc_tiling_on_sc=False,
                                             needs_layout_passes=False),
        scratch_types=[
            pltpu.VMEM((rows_w,), jnp.int32),
            pltpu.VMEM((rows_w,), jnp.int32),
            pltpu.VMEM((4, _SUB, 16), jnp.float32),
            pltpu.VMEM((4, _SUB, 32), jnp.float32),
            pltpu.VMEM((4, 48, _SUB), jnp.float32),
            pltpu.SemaphoreType.DMA,
        ] + [pltpu.SemaphoreType.DMA] * 12,
    )
    def sc_kernel(fdi_h, fti_h, fd_h, ft_h, x_h,
                  fdi_v, fti_v, bfd, bft, tb,
                  semi, *sems):
        iota = lax.iota(jnp.int32, _LANES)
        rows_l = [iota + (g * _LANES) for g in range(_SUB // _LANES)]
        wid = lax.axis_index("s") * 2 + lax.axis_index("c")
        base_c = wid * ch_w
        semgf = sems[0:4]
        semgt = sems[4:8]
        semw = sems[8:12]

        pltpu.async_copy(fdi_h.at[pl.ds(wid * rows_w, rows_w)], fdi_v, semi).wait()
        pltpu.async_copy(fti_h.at[pl.ds(wid * rows_w, rows_w)], fti_v, semi).wait()

        def issue(ci, b):
            s = pl.ds(ci * _SUB, _SUB)
            pltpu.async_copy(fd_h.at[fdi_v.at[s]], bfd.at[b], semgf[b])
            pltpu.async_copy(ft_h.at[fti_v.at[s]], bft.at[b], semgt[b])

        def wait_gather(ci, b):
            s = pl.ds(ci * _SUB, _SUB)
            pltpu.make_async_copy(fd_h.at[fdi_v.at[s]], bfd.at[b], semgf[b]).wait()
            pltpu.make_async_copy(ft_h.at[fti_v.at[s]], bft.at[b], semgt[b]).wait()

        def wait_wb(ci, b):
            pltpu.make_async_copy(tb.at[b], x_h.at[base_c + ci], semw[b]).wait()

        zero = jnp.zeros((_LANES,), jnp.float32)
        for b in range(4):
            for f in list(range(13, 16)) + list(range(41, 48)):
                for g in range(_SUB // _LANES):
                    tb[b, f, pl.ds(g * _LANES, _LANES)] = zero

        for b in range(4):
            issue(b, b)

        def pair(pi, carry):
            for b in range(4):
                ci = pi * 4 + b
                wait_gather(ci, b)

                @pl.when(ci >= 4)
                def _():
                    wait_wb(ci - 4, b)

                for f in range(13):
                    cols = jnp.full((_LANES,), f, jnp.int32)
                    vs = [plsc.load_gather(bfd.at[b], [rows_l[g], cols])
                          for g in range(_SUB // _LANES)]
                    for g in range(_SUB // _LANES):
                        tb[b, f, pl.ds(g * _LANES, _LANES)] = vs[g]
                for f in range(25):
                    cols = jnp.full((_LANES,), f, jnp.int32)
                    vs = [plsc.load_gather(bft.at[b], [rows_l[g], cols])
                          for g in range(_SUB // _LANES)]
                    for g in range(_SUB // _LANES):
                        tb[b, 16 + f, pl.ds(g * _LANES, _LANES)] = vs[g]

                pltpu.async_copy(tb.at[b], x_h.at[base_c + ci], semw[b])

                @pl.when(ci + 4 < ch_w)
                def _():
                    issue(ci + 4, b)
            return carry

        lax.fori_loop(0, ch_w // 4, pair, 0)
        for b in range(4):
            wait_wb(ch_w - 4 + b, b)

    return sc_kernel(fdi, fti, fd_tab, ft_tab)


def _tc_body(x_ref, f1_ref, f2_ref, a1_ref, a2_ref, w2_ref,
             b1_ref, b2_ref, feat_ref, wk_ref, dr_ref, tr_ref):
    r = _BLC * _SUB
    x = x_ref[...].transpose(1, 0, 2).reshape(48, r)
    x1 = x[0:16]
    x2 = x[16:48]
    h1 = (jnp.dot(a1_ref[...], x1, preferred_element_type=jnp.float32)
          + jnp.dot(a2_ref[...], x2, preferred_element_type=jnp.float32)
          + b1_ref[...])
    a = jnp.maximum(h1, 0.01 * h1)
    h2 = jnp.dot(w2_ref[...], a, preferred_element_type=jnp.float32) + b2_ref[...]
    nl = _BLC // 32
    feat_ref[:, 0:1, :] = f1_ref[...].reshape(1, nl, 4096).transpose(1, 0, 2)
    feat_ref[:, 1:2, :] = f2_ref[...].reshape(1, nl, 4096).transpose(1, 0, 2)
    feat_ref[:, 2:7, :] = x2[20:25].reshape(5, nl, 4096).transpose(1, 0, 2)
    feat_ref[:, 7:20, :] = (h2[0:13] + x1[0:13]).reshape(13, nl, 4096).transpose(1, 0, 2)
    feat_ref[:, 20:40, :] = (h2[13:33] + x2[0:20]).reshape(20, nl, 4096).transpose(1, 0, 2)
    wk_ref[...] = x1[0:3].reshape(3, nl, 4096)
    dr_ref[...] = x1[3:13].reshape(10, nl, 4096)
    tr_ref[...] = x2[0:20].reshape(20, nl, 4096)


def _tc_call(x, f1t, f2t, a1, a2, w2, b1c, b2c):
    nch = x.shape[0]
    grid = (nch // _BLC,)
    nl = _BLC // 32
    full = lambda a: pl.BlockSpec(a.shape, lambda i: (0,) * a.ndim)
    out3 = lambda d: pl.BlockSpec((d, nl, 4096), lambda i: (0, i, 0))
    feat_spec = pl.BlockSpec((nl, 40, 4096), lambda i: (i, 0, 0))
    return pl.pallas_call(
        _tc_body,
        grid=grid,
        in_specs=[
            pl.BlockSpec((_BLC, 48, _SUB), lambda i: (i, 0, 0)),
            pl.BlockSpec((nl, 4096), lambda i: (i, 0)),
            pl.BlockSpec((nl, 4096), lambda i: (i, 0)),
            full(a1), full(a2), full(w2), full(b1c), full(b2c),
        ],
        out_specs=[feat_spec, out3(3), out3(10), out3(20)],
        out_shape=[
            jax.ShapeDtypeStruct((200, 40, 4096), jnp.float32),
            jax.ShapeDtypeStruct((3, 200, 4096), jnp.float32),
            jax.ShapeDtypeStruct((10, 200, 4096), jnp.float32),
            jax.ShapeDtypeStruct((20, 200, 4096), jnp.float32),
        ],
    )(x, f1t, f2t, a1, a2, w2, b1c, b2c)


def kernel(links, hw_tab, wk_tab, dt_tab, tm_tab, W1, b1, W2, b2, args):
    bb, ll, _ = links.shape
    n = bb * ll

    i0 = links[:, :, 0].astype(jnp.int32).T.reshape(n)
    i3 = links[:, :, 3].astype(jnp.int32).T.reshape(n)
    i4 = links[:, :, 4].astype(jnp.int32).T.reshape(n)
    i5 = links[:, :, 5].astype(jnp.int32).T.reshape(n)
    f1t = links[:, :, 1].T
    f2t = links[:, :, 2].T
    fdi = i3 * 367 + i4
    fti = i0 * 1441 + i5

    hw_z = hw_tab.at[0].set(0.0)
    nw, nd = wk_tab.shape[0], dt_tab.shape[0]
    nh, nt = hw_tab.shape[0], tm_tab.shape[0]
    fd_tab = jnp.concatenate([
        jnp.broadcast_to(wk_tab[:, None, :], (nw, nd, 3)),
        jnp.broadcast_to(dt_tab[None, :, :], (nw, nd, 10)),
        jnp.zeros((nw, nd, 3), jnp.float32),
    ], axis=-1).reshape(nw * nd, 16)
    ft_tab = jnp.concatenate([
        jnp.broadcast_to(tm_tab[None, :, :], (nh, nt, 20)),
        jnp.broadcast_to(hw_z[:, None, :], (nh, nt, 5)),
        jnp.zeros((nh, nt, 7), jnp.float32),
    ], axis=-1).reshape(nh * nt, 32)

    x = _sc_gather_call(fdi, fti, fd_tab, ft_tab)

    a1 = jnp.concatenate([W1[:, 0:13], jnp.zeros((33, 3), jnp.float32)], 1)
    a2 = jnp.concatenate([W1[:, 13:33], jnp.zeros((33, 12), jnp.float32)], 1)
    featt, wkt, drt, trt = _tc_call(x, f1t, f2t, a1, a2, W2,
                                    b1.reshape(33, 1), b2.reshape(33, 1))
    return (featt.transpose(2, 0, 1),
            (wkt.transpose(2, 1, 0), drt.transpose(2, 1, 0),
             trt.transpose(2, 1, 0)))
